# Initial kernel scaffold; baseline (speedup 1.0000x reference)
import jax
import jax.numpy as jnp
from jax.experimental import pallas as pl


def kernel(x_i, x_j, edge_type, rel_emb):
    raise NotImplementedError("write your pallas kernel here")


# speedup vs baseline: 2.5166x; 2.5166x over previous
import jax
import jax.numpy as jnp
from jax.experimental import pallas as pl
from jax.experimental.pallas import tpu as pltpu

_BE = 1280
_RPAD = 512


def _tc_body(t_ref, rel_ref, xi_ref, xj_ref, out_ref):
    t = t_ref[0, 0, :].astype(jnp.int32)
    iota = jax.lax.broadcasted_iota(jnp.int32, (t.shape[0], _RPAD), 1)
    onehot = (t[:, None] == iota).astype(jnp.bfloat16)
    r = jnp.dot(onehot, rel_ref[...],
                preferred_element_type=jnp.float32)
    p = xi_ref[...] * xj_ref[...]
    out_ref[...] = jnp.sum(p * r, axis=1, keepdims=True)


def kernel(x_i, x_j, edge_type, rel_emb):
    E, D = x_i.shape
    nb = E // _BE
    t3 = edge_type.astype(jnp.int32).reshape(nb, 1, _BE)
    relp = jnp.zeros((_RPAD, D), jnp.bfloat16).at[: rel_emb.shape[0]].set(
        rel_emb.astype(jnp.bfloat16))
    out = pl.pallas_call(
        _tc_body,
        grid=(nb,),
        in_specs=[
            pl.BlockSpec((1, 1, _BE), lambda i: (i, 0, 0)),
            pl.BlockSpec((_RPAD, D), lambda i: (0, 0)),
            pl.BlockSpec((_BE, D), lambda i: (i, 0)),
            pl.BlockSpec((_BE, D), lambda i: (i, 0)),
        ],
        out_specs=pl.BlockSpec((_BE, 1), lambda i: (i, 0)),
        out_shape=jax.ShapeDtypeStruct((E, 1), jnp.float32),
        compiler_params=pltpu.CompilerParams(
            dimension_semantics=("arbitrary",)),
    )(t3, relp, x_i, x_j)
    return out.reshape(E)
